# Initial kernel scaffold; baseline (speedup 1.0000x reference)
import jax
import jax.numpy as jnp
from jax.experimental import pallas as pl


def kernel(x, edge_index, edge_attr, W, att_src, att_dst, W_edge, att_edge, bias, W1, b1, W2, b2):
    raise NotImplementedError("write your pallas kernel here")


# speedup vs baseline: 23.1391x; 23.1391x over previous
import functools

import jax
import jax.numpy as jnp
from jax import lax
from jax.experimental import pallas as pl
from jax.experimental.pallas import tpu as pltpu
from jax.experimental.pallas import tpu_sc as plsc

BA = 2048
CH = 8
K = CH * 128
NSUB = 16
NCORE = 2


def _prep_call(xp, W, att_src, att_dst, Np):
    def body(x_ref, w_ref, asr_ref, adr_ref, t_ref):
        xb = x_ref[...]
        h = jnp.dot(xb, w_ref[...], preferred_element_type=jnp.float32)
        asrc = jnp.sum(h * asr_ref[...][None, :], axis=1, keepdims=True)
        adst = jnp.sum(h * adr_ref[...][None, :], axis=1, keepdims=True)
        t_ref[...] = jnp.concatenate(
            [h, asrc, adst, jnp.zeros((BA, 1), jnp.float32)], axis=1)

    return pl.pallas_call(
        body,
        grid=(Np // BA,),
        in_specs=[
            pl.BlockSpec((BA, 5), lambda i: (i, 0)),
            pl.BlockSpec((5, 5), lambda i: (0, 0)),
            pl.BlockSpec((5,), lambda i: (0,)),
            pl.BlockSpec((5,), lambda i: (0,)),
        ],
        out_specs=pl.BlockSpec((BA, 8), lambda i: (i, 0)),
        out_shape=jax.ShapeDtypeStruct((Np, 8), jnp.float32),
    )(xp, W, att_src, att_dst)


def _sc_edge_pass(srcp, dstp, eap, T, pv, Np, rows_pad):
    STRIPE = Np // NSUB
    LPT = rows_pad // (NCORE * NSUB)
    NCH = LPT // CH

    def body(src_hbm, dst_hbm, ea_hbm, t_hbm, pv_hbm,
             acc_out,
             acc_sh,
             srcv, dstv, eav, tsv, adv, contribv, cntv, pvv,
             gsem, ssem):
        c = lax.axis_index("c")
        s = lax.axis_index("s")
        w = c * NSUB + s
        sbase = pl.multiple_of(s * STRIPE, 128)

        pltpu.sync_copy(pv_hbm, pvv)

        iota16 = lax.iota(jnp.int32, 16)
        zf = jnp.zeros((16,), jnp.float32)

        def zbody(i, carry):
            rows = i * 16 + iota16
            for col in range(8):
                ci = jnp.full((16,), col, jnp.int32)
                plsc.store_scatter(contribv, [rows, ci], zf)
                plsc.store_scatter(cntv, [rows, ci], zf)
            return carry
        lax.fori_loop(0, K // 16, zbody, 0)

        off = 0
        rem = STRIPE
        while rem > 0:
            sz = min(rem, K)
            pltpu.sync_copy(contribv.at[pl.ds(0, sz)],
                            acc_sh.at[pl.ds(sbase + off, sz)])
            off += sz
            rem -= sz

        plsc.subcore_barrier()

        zi = jnp.zeros((16,), jnp.int32)
        oi = jnp.full((16,), 1, jnp.int32)
        v0 = plsc.load_gather(pvv, [zi])
        v1 = plsc.load_gather(pvv, [oi])

        tstart = w * LPT

        def chunk_body(t, carry):
            row_base = tstart + t * CH
            base_e = pl.multiple_of(row_base * 128, K)
            pltpu.sync_copy(src_hbm.at[pl.ds(row_base, CH)], srcv)
            pltpu.sync_copy(dst_hbm.at[pl.ds(row_base, CH)], dstv)
            pltpu.sync_copy(ea_hbm.at[pl.ds(base_e, K)], eav)
            for j in range(CH):
                cp1 = pltpu.async_copy(
                    t_hbm.at[srcv.at[j]], tsv.at[pl.ds(j * 128, 128)], gsem)
                cp2 = pltpu.async_copy(
                    t_hbm.at[dstv.at[j]], adv.at[pl.ds(j * 128, 128)], gsem)
                cp1.wait()
                cp2.wait()

            def ebody(j2, ecarry):
                rows = j2 * 16 + iota16
                asrc = plsc.load_gather(
                    tsv, [rows, jnp.full((16,), 5, jnp.int32)])
                adst = plsc.load_gather(
                    adv, [rows, jnp.full((16,), 6, jnp.int32)])
                dist = plsc.load_gather(eav, [rows, zi])
                cev = plsc.load_gather(eav, [rows, oi])
                alpha = asrc + adst + v0 * dist + v1 * cev
                alpha = jnp.where(alpha >= 0.0, alpha, 0.2 * alpha)
                wgt = jnp.exp(alpha)
                for col in range(5):
                    ci = jnp.full((16,), col, jnp.int32)
                    hc = plsc.load_gather(tsv, [rows, ci])
                    plsc.store_scatter(contribv, [rows, ci], wgt * hc)
                plsc.store_scatter(
                    contribv, [rows, jnp.full((16,), 5, jnp.int32)], wgt)
                below = jnp.where(dist < 0.1, 1.0, 0.0)
                isce = jnp.where(cev == 1.0, 1.0, 0.0)
                plsc.store_scatter(
                    cntv, [rows, jnp.full((16,), 6, jnp.int32)], below)
                plsc.store_scatter(
                    cntv, [rows, jnp.full((16,), 7, jnp.int32)], isce)
                return ecarry
            lax.fori_loop(0, K // 16, ebody, 0)

            for j in range(CH):
                cp1 = pltpu.async_copy(
                    contribv.at[pl.ds(j * 128, 128)],
                    acc_sh.at[dstv.at[j]], ssem, add=True)
                cp2 = pltpu.async_copy(
                    cntv.at[pl.ds(j * 128, 128)],
                    acc_sh.at[srcv.at[j]], ssem, add=True)
                cp1.wait()
                cp2.wait()
            return carry
        lax.fori_loop(0, NCH, chunk_body, 0)

        plsc.subcore_barrier()

        off = 0
        rem = STRIPE
        while rem > 0:
            sz = min(rem, K)
            pltpu.sync_copy(acc_sh.at[pl.ds(sbase + off, sz)],
                            contribv.at[pl.ds(0, sz)])
            pltpu.sync_copy(contribv.at[pl.ds(0, sz)],
                            acc_out.at[c].at[pl.ds(sbase + off, sz)])
            off += sz
            rem -= sz

    mesh = plsc.VectorSubcoreMesh(
        core_axis_name="c", subcore_axis_name="s",
        num_cores=NCORE, num_subcores=NSUB)
    call = pl.kernel(
        body,
        out_type=jax.ShapeDtypeStruct((NCORE, Np, 8), jnp.float32),
        mesh=mesh,
        compiler_params=pltpu.CompilerParams(
            needs_layout_passes=False, use_tc_tiling_on_sc=False),
        scratch_types=[
            pltpu.VMEM_SHARED((Np, 8), jnp.float32),
            pltpu.VMEM((CH, 128), jnp.int32),
            pltpu.VMEM((CH, 128), jnp.int32),
            pltpu.VMEM((K, 2), jnp.float32),
            pltpu.VMEM((K, 8), jnp.float32),
            pltpu.VMEM((K, 8), jnp.float32),
            pltpu.VMEM((K, 8), jnp.float32),
            pltpu.VMEM((K, 8), jnp.float32),
            pltpu.VMEM((16,), jnp.float32),
            pltpu.SemaphoreType.DMA,
            pltpu.SemaphoreType.DMA,
        ],
    )
    return call(srcp, dstp, eap, T, pv)


def _post_call(acc, xp, bias, W1, b1, W2, b2, Np, n):
    def body(acc_ref, x_ref, bias_ref, w1_ref, b1_ref, w2_ref,
             b2_ref, newx_ref, vel_ref, bc_ref, fr_ref, dc_ref):
        i = pl.program_id(0)
        a = acc_ref[0] + acc_ref[1]
        num = a[:, 0:5]
        den = a[:, 5:6]
        out = num / (den + 1e-16) + bias_ref[...][None, :]
        h2 = jnp.maximum(
            jnp.dot(out, w1_ref[...], preferred_element_type=jnp.float32)
            + b1_ref[...][None, :], 0.0)
        h2 = jnp.maximum(
            jnp.dot(h2, w2_ref[...], preferred_element_type=jnp.float32)
            + b2_ref[...][None, :], 0.0)
        h2 = h2 * 2.0 - 1.0
        xb = x_ref[...]
        x4 = xb[:, 4:5]
        food = jnp.where(x4 == 1.0, 1.0, 0.0)
        accv = h2 * 0.01 * food
        velo = jnp.clip(xb[:, 2:4] + accv, -0.1, 0.1)
        posn = xb[:, 0:2] + velo
        newx_ref[...] = jnp.concatenate([posn, velo, x4], axis=1)
        vel_ref[...] = velo
        rowid = lax.broadcasted_iota(jnp.int32, (BA, 1), 0) + i * BA
        valid = jnp.where(rowid < n, 1.0, 0.0)
        absx = jnp.abs(posn[:, 0:1])
        absy = jnp.abs(posn[:, 1:2])
        bx = jnp.log(absx + 1e-6) * jnp.where(absx > 1.0, 1.0, 0.0)
        by = jnp.log(absy + 1e-6) * jnp.where(absy > 1.0, 1.0, 0.0)
        bpart = jnp.sum((bx + by) * valid)
        consume = jnp.where((x4 == 0.0) & (a[:, 6:7] >= 3.0), 1.0, 0.0) * valid
        deadv = jnp.where((x4 == 1.0) & (a[:, 7:8] < 1.0), 1.0, 0.0) * valid
        fpart = jnp.sum(consume)
        dpart = jnp.sum(deadv)

        @pl.when(i == 0)
        def _():
            bc_ref[0, 0] = 0.0
            fr_ref[0, 0] = 0.0
            dc_ref[0, 0] = 0.0
        bc_ref[0, 0] += bpart
        fr_ref[0, 0] += fpart
        dc_ref[0, 0] += dpart

    return pl.pallas_call(
        body,
        grid=(Np // BA,),
        in_specs=[
            pl.BlockSpec((2, BA, 8), lambda i: (0, i, 0)),
            pl.BlockSpec((BA, 5), lambda i: (i, 0)),
            pl.BlockSpec((5,), lambda i: (0,)),
            pl.BlockSpec((5, 5), lambda i: (0, 0)),
            pl.BlockSpec((5,), lambda i: (0,)),
            pl.BlockSpec((5, 2), lambda i: (0, 0)),
            pl.BlockSpec((2,), lambda i: (0,)),
        ],
        out_specs=[
            pl.BlockSpec((BA, 5), lambda i: (i, 0)),
            pl.BlockSpec((BA, 2), lambda i: (i, 0)),
            pl.BlockSpec((1, 1), lambda i: (0, 0), memory_space=pltpu.SMEM),
            pl.BlockSpec((1, 1), lambda i: (0, 0), memory_space=pltpu.SMEM),
            pl.BlockSpec((1, 1), lambda i: (0, 0), memory_space=pltpu.SMEM),
        ],
        out_shape=[
            jax.ShapeDtypeStruct((Np, 5), jnp.float32),
            jax.ShapeDtypeStruct((Np, 2), jnp.float32),
            jax.ShapeDtypeStruct((1, 1), jnp.float32),
            jax.ShapeDtypeStruct((1, 1), jnp.float32),
            jax.ShapeDtypeStruct((1, 1), jnp.float32),
        ],
    )(acc, xp, bias, W1, b1, W2, b2)


def kernel(x, edge_index, edge_attr, W, att_src, att_dst, W_edge, att_edge,
           bias, W1, b1, W2, b2):
    n = x.shape[0]
    e = edge_index.shape[1]
    Np = ((n + 1 + BA - 1) // BA) * BA
    rows = (e + 127) // 128
    rows_pad = ((rows + NCORE * NSUB * CH - 1)
                // (NCORE * NSUB * CH)) * (NCORE * NSUB * CH)
    Ep = rows_pad * 128
    pad_e = Ep - e

    xp = jnp.zeros((Np, 5), jnp.float32).at[:n, :].set(x)
    src = edge_index[0]
    dst = edge_index[1]
    dummy = jnp.full((pad_e,), n, jnp.int32)
    srcp = jnp.concatenate([src, dummy]).reshape(rows_pad, 128)
    dstp = jnp.concatenate([dst, dummy]).reshape(rows_pad, 128)
    eap = jnp.concatenate(
        [edge_attr.astype(jnp.float32), jnp.zeros((pad_e, 2), jnp.float32)],
        axis=0)
    v = W_edge @ att_edge
    pv = jnp.concatenate([v, jnp.zeros((14,), jnp.float32)]).astype(jnp.float32)

    T = _prep_call(xp, W, att_src, att_dst, Np)
    acc = _sc_edge_pass(srcp, dstp, eap, T, pv, Np, rows_pad)
    newx, vel, bc, fr, dc = _post_call(acc, xp, bias, W1, b1, W2, b2, Np, n)
    return (newx[:n], vel[:n], bc.reshape(()), fr.reshape(()),
            dc.reshape(()))


# speedup vs baseline: 117.7546x; 5.0890x over previous
import functools

import jax
import jax.numpy as jnp
from jax import lax
from jax.experimental import pallas as pl
from jax.experimental.pallas import tpu as pltpu
from jax.experimental.pallas import tpu_sc as plsc

BA = 2048
CH = 8
K = CH * 128
NSUB = 16
NCORE = 2


def _prep_call(xp, W, att_src, att_dst, Np):
    def body(x_ref, w_ref, asr_ref, adr_ref, t_ref):
        xb = x_ref[...]
        h = jnp.dot(xb, w_ref[...], preferred_element_type=jnp.float32)
        asrc = jnp.sum(h * asr_ref[...][None, :], axis=1, keepdims=True)
        adst = jnp.sum(h * adr_ref[...][None, :], axis=1, keepdims=True)
        t_ref[...] = jnp.concatenate(
            [h, asrc, adst, jnp.zeros((BA, 1), jnp.float32)], axis=1)

    return pl.pallas_call(
        body,
        grid=(Np // BA,),
        in_specs=[
            pl.BlockSpec((BA, 5), lambda i: (i, 0)),
            pl.BlockSpec((5, 5), lambda i: (0, 0)),
            pl.BlockSpec((5,), lambda i: (0,)),
            pl.BlockSpec((5,), lambda i: (0,)),
        ],
        out_specs=pl.BlockSpec((BA, 8), lambda i: (i, 0)),
        out_shape=jax.ShapeDtypeStruct((Np, 8), jnp.float32),
    )(xp, W, att_src, att_dst)


def _sc_edge_pass(srcp, dstp, dist2, ce2, T, pv, Np, rows_pad):
    STRIPE = Np // NSUB
    LPT = rows_pad // (NCORE * NSUB)
    NCH = LPT // CH

    def body(src_hbm, dst_hbm, dist_hbm, ce_hbm, t_hbm, pv_hbm,
             acc_out,
             acc_sh,
             srcv, dstv, distv, cevv, tsv, adv, contribv, cntv, pvv,
             gsem, ssem):
        c = lax.axis_index("c")
        s = lax.axis_index("s")
        w = c * NSUB + s
        sbase = pl.multiple_of(s * STRIPE, 128)

        pltpu.sync_copy(pv_hbm, pvv)

        iota16 = lax.iota(jnp.int32, 16)
        zf = jnp.zeros((16,), jnp.float32)

        def zbody(i, carry):
            rows = i * 16 + iota16
            for col in range(8):
                ci = jnp.full((16,), col, jnp.int32)
                plsc.store_scatter(contribv, [rows, ci], zf)
                plsc.store_scatter(cntv, [rows, ci], zf)
            return carry
        lax.fori_loop(0, K // 16, zbody, 0)

        off = 0
        rem = STRIPE
        while rem > 0:
            sz = min(rem, K)
            pltpu.sync_copy(contribv.at[pl.ds(0, sz)],
                            acc_sh.at[pl.ds(sbase + off, sz)])
            off += sz
            rem -= sz

        plsc.subcore_barrier()

        zi = jnp.zeros((16,), jnp.int32)
        oi = jnp.full((16,), 1, jnp.int32)
        v0 = plsc.load_gather(pvv, [zi])
        v1 = plsc.load_gather(pvv, [oi])

        tstart = w * LPT

        def chunk_body(t, carry):
            row_base = tstart + t * CH
            pltpu.sync_copy(src_hbm.at[pl.ds(row_base, CH)], srcv)
            pltpu.sync_copy(dst_hbm.at[pl.ds(row_base, CH)], dstv)
            pltpu.sync_copy(dist_hbm.at[pl.ds(row_base, CH)], distv)
            pltpu.sync_copy(ce_hbm.at[pl.ds(row_base, CH)], cevv)
            cps = []
            for j in range(CH):
                cps.append(pltpu.async_copy(
                    t_hbm.at[srcv.at[j]], tsv.at[pl.ds(j * 128, 128)], gsem))
                cps.append(pltpu.async_copy(
                    t_hbm.at[dstv.at[j]], adv.at[pl.ds(j * 128, 128)], gsem))
            for cp in cps:
                cp.wait()

            def ebody(j, ecarry):
                jr = jnp.full((16,), j, jnp.int32)
                for q in range(8):
                    cq = q * 16 + iota16
                    rows = j * 128 + cq
                    asrc = plsc.load_gather(
                        tsv, [rows, jnp.full((16,), 5, jnp.int32)])
                    adst = plsc.load_gather(
                        adv, [rows, jnp.full((16,), 6, jnp.int32)])
                    dist = plsc.load_gather(distv, [jr, cq])
                    cev = plsc.load_gather(cevv, [jr, cq])
                    alpha = asrc + adst + v0 * dist + v1 * cev
                    alpha = jnp.where(alpha >= 0.0, alpha, 0.2 * alpha)
                    wgt = jnp.exp(alpha)
                    for col in range(5):
                        ci = jnp.full((16,), col, jnp.int32)
                        hc = plsc.load_gather(tsv, [rows, ci])
                        plsc.store_scatter(contribv, [rows, ci], wgt * hc)
                    plsc.store_scatter(
                        contribv, [rows, jnp.full((16,), 5, jnp.int32)], wgt)
                    below = jnp.where(dist < 0.1, 1.0, 0.0)
                    isce = jnp.where(cev == 1.0, 1.0, 0.0)
                    plsc.store_scatter(
                        cntv, [rows, jnp.full((16,), 6, jnp.int32)], below)
                    plsc.store_scatter(
                        cntv, [rows, jnp.full((16,), 7, jnp.int32)], isce)
                return ecarry
            lax.fori_loop(0, CH, ebody, 0)

            scps = []
            for j in range(CH):
                scps.append(pltpu.async_copy(
                    contribv.at[pl.ds(j * 128, 128)],
                    acc_sh.at[dstv.at[j]], ssem, add=True))
                scps.append(pltpu.async_copy(
                    cntv.at[pl.ds(j * 128, 128)],
                    acc_sh.at[srcv.at[j]], ssem, add=True))
            for cp in scps:
                cp.wait()
            return carry
        lax.fori_loop(0, NCH, chunk_body, 0)

        plsc.subcore_barrier()

        off = 0
        rem = STRIPE
        while rem > 0:
            sz = min(rem, K)
            pltpu.sync_copy(acc_sh.at[pl.ds(sbase + off, sz)],
                            contribv.at[pl.ds(0, sz)])
            pltpu.sync_copy(contribv.at[pl.ds(0, sz)],
                            acc_out.at[c].at[pl.ds(sbase + off, sz)])
            off += sz
            rem -= sz

    mesh = plsc.VectorSubcoreMesh(
        core_axis_name="c", subcore_axis_name="s",
        num_cores=NCORE, num_subcores=NSUB)
    call = pl.kernel(
        body,
        out_type=jax.ShapeDtypeStruct((NCORE, Np, 8), jnp.float32),
        mesh=mesh,
        compiler_params=pltpu.CompilerParams(
            needs_layout_passes=False, use_tc_tiling_on_sc=False),
        scratch_types=[
            pltpu.VMEM_SHARED((Np, 8), jnp.float32),
            pltpu.VMEM((CH, 128), jnp.int32),
            pltpu.VMEM((CH, 128), jnp.int32),
            pltpu.VMEM((CH, 128), jnp.float32),
            pltpu.VMEM((CH, 128), jnp.float32),
            pltpu.VMEM((K, 8), jnp.float32),
            pltpu.VMEM((K, 8), jnp.float32),
            pltpu.VMEM((K, 8), jnp.float32),
            pltpu.VMEM((K, 8), jnp.float32),
            pltpu.VMEM((16,), jnp.float32),
            pltpu.SemaphoreType.DMA,
            pltpu.SemaphoreType.DMA,
        ],
    )
    return call(srcp, dstp, dist2, ce2, T, pv)


def _post_call(acc, xp, bias, W1, b1, W2, b2, Np, n):
    def body(acc_ref, x_ref, bias_ref, w1_ref, b1_ref, w2_ref,
             b2_ref, newx_ref, vel_ref, bc_ref, fr_ref, dc_ref):
        i = pl.program_id(0)
        a = acc_ref[0] + acc_ref[1]
        num = a[:, 0:5]
        den = a[:, 5:6]
        out = num / (den + 1e-16) + bias_ref[...][None, :]
        h2 = jnp.maximum(
            jnp.dot(out, w1_ref[...], preferred_element_type=jnp.float32)
            + b1_ref[...][None, :], 0.0)
        h2 = jnp.maximum(
            jnp.dot(h2, w2_ref[...], preferred_element_type=jnp.float32)
            + b2_ref[...][None, :], 0.0)
        h2 = h2 * 2.0 - 1.0
        xb = x_ref[...]
        x4 = xb[:, 4:5]
        food = jnp.where(x4 == 1.0, 1.0, 0.0)
        accv = h2 * 0.01 * food
        velo = jnp.clip(xb[:, 2:4] + accv, -0.1, 0.1)
        posn = xb[:, 0:2] + velo
        newx_ref[...] = jnp.concatenate([posn, velo, x4], axis=1)
        vel_ref[...] = velo
        rowid = lax.broadcasted_iota(jnp.int32, (BA, 1), 0) + i * BA
        valid = jnp.where(rowid < n, 1.0, 0.0)
        absx = jnp.abs(posn[:, 0:1])
        absy = jnp.abs(posn[:, 1:2])
        bx = jnp.log(absx + 1e-6) * jnp.where(absx > 1.0, 1.0, 0.0)
        by = jnp.log(absy + 1e-6) * jnp.where(absy > 1.0, 1.0, 0.0)
        bpart = jnp.sum((bx + by) * valid)
        consume = jnp.where((x4 == 0.0) & (a[:, 6:7] >= 3.0), 1.0, 0.0) * valid
        deadv = jnp.where((x4 == 1.0) & (a[:, 7:8] < 1.0), 1.0, 0.0) * valid
        fpart = jnp.sum(consume)
        dpart = jnp.sum(deadv)

        @pl.when(i == 0)
        def _():
            bc_ref[0, 0] = 0.0
            fr_ref[0, 0] = 0.0
            dc_ref[0, 0] = 0.0
        bc_ref[0, 0] += bpart
        fr_ref[0, 0] += fpart
        dc_ref[0, 0] += dpart

    return pl.pallas_call(
        body,
        grid=(Np // BA,),
        in_specs=[
            pl.BlockSpec((2, BA, 8), lambda i: (0, i, 0)),
            pl.BlockSpec((BA, 5), lambda i: (i, 0)),
            pl.BlockSpec((5,), lambda i: (0,)),
            pl.BlockSpec((5, 5), lambda i: (0, 0)),
            pl.BlockSpec((5,), lambda i: (0,)),
            pl.BlockSpec((5, 2), lambda i: (0, 0)),
            pl.BlockSpec((2,), lambda i: (0,)),
        ],
        out_specs=[
            pl.BlockSpec((BA, 5), lambda i: (i, 0)),
            pl.BlockSpec((BA, 2), lambda i: (i, 0)),
            pl.BlockSpec((1, 1), lambda i: (0, 0), memory_space=pltpu.SMEM),
            pl.BlockSpec((1, 1), lambda i: (0, 0), memory_space=pltpu.SMEM),
            pl.BlockSpec((1, 1), lambda i: (0, 0), memory_space=pltpu.SMEM),
        ],
        out_shape=[
            jax.ShapeDtypeStruct((Np, 5), jnp.float32),
            jax.ShapeDtypeStruct((Np, 2), jnp.float32),
            jax.ShapeDtypeStruct((1, 1), jnp.float32),
            jax.ShapeDtypeStruct((1, 1), jnp.float32),
            jax.ShapeDtypeStruct((1, 1), jnp.float32),
        ],
    )(acc, xp, bias, W1, b1, W2, b2)


def kernel(x, edge_index, edge_attr, W, att_src, att_dst, W_edge, att_edge,
           bias, W1, b1, W2, b2):
    n = x.shape[0]
    e = edge_index.shape[1]
    Np = ((n + 1 + BA - 1) // BA) * BA
    rows = (e + 127) // 128
    rows_pad = ((rows + NCORE * NSUB * CH - 1)
                // (NCORE * NSUB * CH)) * (NCORE * NSUB * CH)
    Ep = rows_pad * 128
    pad_e = Ep - e

    xp = jnp.zeros((Np, 5), jnp.float32).at[:n, :].set(x)
    src = edge_index[0]
    dst = edge_index[1]
    dummy = jnp.full((pad_e,), n, jnp.int32)
    srcp = jnp.concatenate([src, dummy]).reshape(rows_pad, 128)
    dstp = jnp.concatenate([dst, dummy]).reshape(rows_pad, 128)
    zpad = jnp.zeros((pad_e,), jnp.float32)
    dist2 = jnp.concatenate(
        [edge_attr[:, 0].astype(jnp.float32), zpad]).reshape(rows_pad, 128)
    ce2 = jnp.concatenate(
        [edge_attr[:, 1].astype(jnp.float32), zpad]).reshape(rows_pad, 128)
    v = W_edge @ att_edge
    pv = jnp.concatenate([v, jnp.zeros((14,), jnp.float32)]).astype(jnp.float32)

    T = _prep_call(xp, W, att_src, att_dst, Np)
    acc = _sc_edge_pass(srcp, dstp, dist2, ce2, T, pv, Np, rows_pad)
    newx, vel, bc, fr, dc = _post_call(acc, xp, bias, W1, b1, W2, b2, Np, n)
    return (newx[:n], vel[:n], bc.reshape(()), fr.reshape(()),
            dc.reshape(()))


# speedup vs baseline: 161.9950x; 1.3757x over previous
import functools

import jax
import jax.numpy as jnp
from jax import lax
from jax.experimental import pallas as pl
from jax.experimental.pallas import tpu as pltpu
from jax.experimental.pallas import tpu_sc as plsc

BA = 2048
CH = 8
K = CH * 128
NSUB = 16
NCORE = 2


def _prep_call(xp, W, att_src, att_dst, Np):
    def body(x_ref, w_ref, asr_ref, adr_ref, t_ref):
        xb = x_ref[...]
        h = jnp.dot(xb, w_ref[...], preferred_element_type=jnp.float32)
        asrc = jnp.sum(h * asr_ref[...][None, :], axis=1, keepdims=True)
        adst = jnp.sum(h * adr_ref[...][None, :], axis=1, keepdims=True)
        t_ref[...] = jnp.concatenate(
            [h, asrc, adst, jnp.zeros((BA, 1), jnp.float32)], axis=1)

    return pl.pallas_call(
        body,
        grid=(Np // BA,),
        in_specs=[
            pl.BlockSpec((BA, 5), lambda i: (i, 0)),
            pl.BlockSpec((5, 5), lambda i: (0, 0)),
            pl.BlockSpec((5,), lambda i: (0,)),
            pl.BlockSpec((5,), lambda i: (0,)),
        ],
        out_specs=pl.BlockSpec((BA, 8), lambda i: (i, 0)),
        out_shape=jax.ShapeDtypeStruct((Np, 8), jnp.float32),
    )(xp, W, att_src, att_dst)


def _sc_edge_pass(srcp, dstp, dist2, ce2, T, pv, Np, rows_pad):
    STRIPE = Np // NSUB
    LPT = rows_pad // (NCORE * NSUB)
    NCH = LPT // CH

    def body(src_hbm, dst_hbm, dist_hbm, ce_hbm, t_hbm, pv_hbm,
             acc_out,
             acc_sh,
             srcv0, dstv0, tsv0, adv0,
             srcv1, dstv1, tsv1, adv1,
             distv, cevv, contribv, cntv, pvv,
             gsem, ssem):
        bufs = [(srcv0, dstv0, tsv0, adv0), (srcv1, dstv1, tsv1, adv1)]
        c = lax.axis_index("c")
        s = lax.axis_index("s")
        w = c * NSUB + s
        sbase = pl.multiple_of(s * STRIPE, 128)

        pltpu.sync_copy(pv_hbm, pvv)

        iota16 = lax.iota(jnp.int32, 16)
        zf = jnp.zeros((16,), jnp.float32)

        def zbody(i, carry):
            rows = i * 16 + iota16
            for col in range(8):
                ci = jnp.full((16,), col, jnp.int32)
                plsc.store_scatter(contribv, [rows, ci], zf)
                plsc.store_scatter(cntv, [rows, ci], zf)
            return carry
        lax.fori_loop(0, K // 16, zbody, 0)

        off = 0
        rem = STRIPE
        while rem > 0:
            sz = min(rem, K)
            pltpu.sync_copy(contribv.at[pl.ds(0, sz)],
                            acc_sh.at[pl.ds(sbase + off, sz)])
            off += sz
            rem -= sz

        plsc.subcore_barrier()

        zi = jnp.zeros((16,), jnp.int32)
        oi = jnp.full((16,), 1, jnp.int32)
        v0 = plsc.load_gather(pvv, [zi])
        v1 = plsc.load_gather(pvv, [oi])

        tstart = w * LPT

        def copy_idx(t, b):
            srcv, dstv = bufs[b][0], bufs[b][1]
            row_base = tstart + t * CH
            pltpu.sync_copy(src_hbm.at[pl.ds(row_base, CH)], srcv)
            pltpu.sync_copy(dst_hbm.at[pl.ds(row_base, CH)], dstv)

        def copy_ea(t):
            row_base = tstart + t * CH
            pltpu.sync_copy(dist_hbm.at[pl.ds(row_base, CH)], distv)
            pltpu.sync_copy(ce_hbm.at[pl.ds(row_base, CH)], cevv)

        def issue_gathers(b):
            srcv, dstv, tsv, adv = bufs[b]
            for j in range(CH):
                pltpu.async_copy(
                    t_hbm.at[srcv.at[j]], tsv.at[pl.ds(j * 128, 128)], gsem)
                pltpu.async_copy(
                    t_hbm.at[dstv.at[j]], adv.at[pl.ds(j * 128, 128)], gsem)

        def wait_gathers(b):
            srcv, dstv, tsv, adv = bufs[b]
            for j in range(CH):
                pltpu.make_async_copy(
                    t_hbm.at[srcv.at[j]], tsv.at[pl.ds(j * 128, 128)],
                    gsem).wait()
                pltpu.make_async_copy(
                    t_hbm.at[dstv.at[j]], adv.at[pl.ds(j * 128, 128)],
                    gsem).wait()

        def issue_scatters(b):
            srcv, dstv, _, _ = bufs[b]
            for j in range(CH):
                pltpu.async_copy(
                    contribv.at[pl.ds(j * 128, 128)],
                    acc_sh.at[dstv.at[j]], ssem, add=True)
                pltpu.async_copy(
                    cntv.at[pl.ds(j * 128, 128)],
                    acc_sh.at[srcv.at[j]], ssem, add=True)

        def wait_scatters(b):
            srcv, dstv, _, _ = bufs[b]
            for j in range(CH):
                pltpu.make_async_copy(
                    contribv.at[pl.ds(j * 128, 128)],
                    acc_sh.at[dstv.at[j]], ssem).wait()
                pltpu.make_async_copy(
                    cntv.at[pl.ds(j * 128, 128)],
                    acc_sh.at[srcv.at[j]], ssem).wait()

        def compute(b):
            _, _, tsv, adv = bufs[b]

            def ebody(j, ecarry):
                jr = jnp.full((16,), j, jnp.int32)
                for q in range(8):
                    cq = q * 16 + iota16
                    rows = j * 128 + cq
                    asrc = plsc.load_gather(
                        tsv, [rows, jnp.full((16,), 5, jnp.int32)])
                    adst = plsc.load_gather(
                        adv, [rows, jnp.full((16,), 6, jnp.int32)])
                    dist = plsc.load_gather(distv, [jr, cq])
                    cev = plsc.load_gather(cevv, [jr, cq])
                    alpha = asrc + adst + v0 * dist + v1 * cev
                    alpha = jnp.where(alpha >= 0.0, alpha, 0.2 * alpha)
                    wgt = jnp.exp(alpha)
                    for col in range(5):
                        ci = jnp.full((16,), col, jnp.int32)
                        hc = plsc.load_gather(tsv, [rows, ci])
                        plsc.store_scatter(contribv, [rows, ci], wgt * hc)
                    plsc.store_scatter(
                        contribv, [rows, jnp.full((16,), 5, jnp.int32)], wgt)
                    below = jnp.where(dist < 0.1, 1.0, 0.0)
                    isce = jnp.where(cev == 1.0, 1.0, 0.0)
                    plsc.store_scatter(
                        cntv, [rows, jnp.full((16,), 6, jnp.int32)], below)
                    plsc.store_scatter(
                        cntv, [rows, jnp.full((16,), 7, jnp.int32)], isce)
                return ecarry
            lax.fori_loop(0, CH, ebody, 0)

        copy_idx(0, 0)
        copy_ea(0)
        issue_gathers(0)

        def pipe_body(i, carry):
            t0 = i * 2
            t1 = t0 + 1

            @pl.when(t0 > 0)
            def _():
                wait_scatters(1)
            copy_idx(t1, 1)
            issue_gathers(1)
            wait_gathers(0)
            compute(0)
            issue_scatters(0)
            copy_ea(t1)

            wait_scatters(0)

            @pl.when(t1 + 1 < NCH)
            def _():
                copy_idx(t1 + 1, 0)
                issue_gathers(0)
            wait_gathers(1)
            compute(1)
            issue_scatters(1)

            @pl.when(t1 + 1 < NCH)
            def _():
                copy_ea(t1 + 1)
            return carry
        lax.fori_loop(0, NCH // 2, pipe_body, 0)
        wait_scatters(1)

        plsc.subcore_barrier()

        off = 0
        rem = STRIPE
        while rem > 0:
            sz = min(rem, K)
            pltpu.sync_copy(acc_sh.at[pl.ds(sbase + off, sz)],
                            contribv.at[pl.ds(0, sz)])
            pltpu.sync_copy(contribv.at[pl.ds(0, sz)],
                            acc_out.at[c].at[pl.ds(sbase + off, sz)])
            off += sz
            rem -= sz

    mesh = plsc.VectorSubcoreMesh(
        core_axis_name="c", subcore_axis_name="s",
        num_cores=NCORE, num_subcores=NSUB)
    call = pl.kernel(
        body,
        out_type=jax.ShapeDtypeStruct((NCORE, Np, 8), jnp.float32),
        mesh=mesh,
        compiler_params=pltpu.CompilerParams(
            needs_layout_passes=False, use_tc_tiling_on_sc=False),
        scratch_types=[
            pltpu.VMEM_SHARED((Np, 8), jnp.float32),
            pltpu.VMEM((CH, 128), jnp.int32),
            pltpu.VMEM((CH, 128), jnp.int32),
            pltpu.VMEM((K, 8), jnp.float32),
            pltpu.VMEM((K, 8), jnp.float32),
            pltpu.VMEM((CH, 128), jnp.int32),
            pltpu.VMEM((CH, 128), jnp.int32),
            pltpu.VMEM((K, 8), jnp.float32),
            pltpu.VMEM((K, 8), jnp.float32),
            pltpu.VMEM((CH, 128), jnp.float32),
            pltpu.VMEM((CH, 128), jnp.float32),
            pltpu.VMEM((K, 8), jnp.float32),
            pltpu.VMEM((K, 8), jnp.float32),
            pltpu.VMEM((16,), jnp.float32),
            pltpu.SemaphoreType.DMA,
            pltpu.SemaphoreType.DMA,
        ],
    )
    return call(srcp, dstp, dist2, ce2, T, pv)


def _post_call(acc, xp, bias, W1, b1, W2, b2, Np, n):
    def body(acc_ref, x_ref, bias_ref, w1_ref, b1_ref, w2_ref,
             b2_ref, newx_ref, vel_ref, bc_ref, fr_ref, dc_ref):
        i = pl.program_id(0)
        a = acc_ref[0] + acc_ref[1]
        num = a[:, 0:5]
        den = a[:, 5:6]
        out = num / (den + 1e-16) + bias_ref[...][None, :]
        h2 = jnp.maximum(
            jnp.dot(out, w1_ref[...], preferred_element_type=jnp.float32)
            + b1_ref[...][None, :], 0.0)
        h2 = jnp.maximum(
            jnp.dot(h2, w2_ref[...], preferred_element_type=jnp.float32)
            + b2_ref[...][None, :], 0.0)
        h2 = h2 * 2.0 - 1.0
        xb = x_ref[...]
        x4 = xb[:, 4:5]
        food = jnp.where(x4 == 1.0, 1.0, 0.0)
        accv = h2 * 0.01 * food
        velo = jnp.clip(xb[:, 2:4] + accv, -0.1, 0.1)
        posn = xb[:, 0:2] + velo
        newx_ref[...] = jnp.concatenate([posn, velo, x4], axis=1)
        vel_ref[...] = velo
        rowid = lax.broadcasted_iota(jnp.int32, (BA, 1), 0) + i * BA
        valid = jnp.where(rowid < n, 1.0, 0.0)
        absx = jnp.abs(posn[:, 0:1])
        absy = jnp.abs(posn[:, 1:2])
        bx = jnp.log(absx + 1e-6) * jnp.where(absx > 1.0, 1.0, 0.0)
        by = jnp.log(absy + 1e-6) * jnp.where(absy > 1.0, 1.0, 0.0)
        bpart = jnp.sum((bx + by) * valid)
        consume = jnp.where((x4 == 0.0) & (a[:, 6:7] >= 3.0), 1.0, 0.0) * valid
        deadv = jnp.where((x4 == 1.0) & (a[:, 7:8] < 1.0), 1.0, 0.0) * valid
        fpart = jnp.sum(consume)
        dpart = jnp.sum(deadv)

        @pl.when(i == 0)
        def _():
            bc_ref[0, 0] = 0.0
            fr_ref[0, 0] = 0.0
            dc_ref[0, 0] = 0.0
        bc_ref[0, 0] += bpart
        fr_ref[0, 0] += fpart
        dc_ref[0, 0] += dpart

    return pl.pallas_call(
        body,
        grid=(Np // BA,),
        in_specs=[
            pl.BlockSpec((2, BA, 8), lambda i: (0, i, 0)),
            pl.BlockSpec((BA, 5), lambda i: (i, 0)),
            pl.BlockSpec((5,), lambda i: (0,)),
            pl.BlockSpec((5, 5), lambda i: (0, 0)),
            pl.BlockSpec((5,), lambda i: (0,)),
            pl.BlockSpec((5, 2), lambda i: (0, 0)),
            pl.BlockSpec((2,), lambda i: (0,)),
        ],
        out_specs=[
            pl.BlockSpec((BA, 5), lambda i: (i, 0)),
            pl.BlockSpec((BA, 2), lambda i: (i, 0)),
            pl.BlockSpec((1, 1), lambda i: (0, 0), memory_space=pltpu.SMEM),
            pl.BlockSpec((1, 1), lambda i: (0, 0), memory_space=pltpu.SMEM),
            pl.BlockSpec((1, 1), lambda i: (0, 0), memory_space=pltpu.SMEM),
        ],
        out_shape=[
            jax.ShapeDtypeStruct((Np, 5), jnp.float32),
            jax.ShapeDtypeStruct((Np, 2), jnp.float32),
            jax.ShapeDtypeStruct((1, 1), jnp.float32),
            jax.ShapeDtypeStruct((1, 1), jnp.float32),
            jax.ShapeDtypeStruct((1, 1), jnp.float32),
        ],
    )(acc, xp, bias, W1, b1, W2, b2)


def kernel(x, edge_index, edge_attr, W, att_src, att_dst, W_edge, att_edge,
           bias, W1, b1, W2, b2):
    n = x.shape[0]
    e = edge_index.shape[1]
    Np = ((n + 1 + BA - 1) // BA) * BA
    rows = (e + 127) // 128
    unit = NCORE * NSUB * CH * 2
    rows_pad = ((rows + unit - 1) // unit) * unit
    Ep = rows_pad * 128
    pad_e = Ep - e

    xp = jnp.zeros((Np, 5), jnp.float32).at[:n, :].set(x)
    src = edge_index[0]
    dst = edge_index[1]
    dummy = jnp.full((pad_e,), n, jnp.int32)
    srcp = jnp.concatenate([src, dummy]).reshape(rows_pad, 128)
    dstp = jnp.concatenate([dst, dummy]).reshape(rows_pad, 128)
    zpad = jnp.zeros((pad_e,), jnp.float32)
    dist2 = jnp.concatenate(
        [edge_attr[:, 0].astype(jnp.float32), zpad]).reshape(rows_pad, 128)
    ce2 = jnp.concatenate(
        [edge_attr[:, 1].astype(jnp.float32), zpad]).reshape(rows_pad, 128)
    v = W_edge @ att_edge
    pv = jnp.concatenate([v, jnp.zeros((14,), jnp.float32)]).astype(jnp.float32)

    T = _prep_call(xp, W, att_src, att_dst, Np)
    acc = _sc_edge_pass(srcp, dstp, dist2, ce2, T, pv, Np, rows_pad)
    newx, vel, bc, fr, dc = _post_call(acc, xp, bias, W1, b1, W2, b2, Np, n)
    return (newx[:n], vel[:n], bc.reshape(()), fr.reshape(()),
            dc.reshape(()))


# speedup vs baseline: 177.6133x; 1.0964x over previous
import functools

import jax
import jax.numpy as jnp
from jax import lax
from jax.experimental import pallas as pl
from jax.experimental.pallas import tpu as pltpu
from jax.experimental.pallas import tpu_sc as plsc

BA = 2048
CH = 8
K = CH * 128
NSUB = 16
NCORE = 2


def _prep_call(xp, W, att_src, att_dst, Np):
    def body(x_ref, w_ref, asr_ref, adr_ref, t_ref):
        xb = x_ref[...]
        h = jnp.dot(xb, w_ref[...], preferred_element_type=jnp.float32)
        asrc = jnp.sum(h * asr_ref[...][None, :], axis=1, keepdims=True)
        adst = jnp.sum(h * adr_ref[...][None, :], axis=1, keepdims=True)
        t_ref[...] = jnp.concatenate(
            [h, asrc, adst, jnp.zeros((BA, 1), jnp.float32)], axis=1)

    return pl.pallas_call(
        body,
        grid=(Np // BA,),
        in_specs=[
            pl.BlockSpec((BA, 5), lambda i: (i, 0)),
            pl.BlockSpec((5, 5), lambda i: (0, 0)),
            pl.BlockSpec((5,), lambda i: (0,)),
            pl.BlockSpec((5,), lambda i: (0,)),
        ],
        out_specs=pl.BlockSpec((BA, 8), lambda i: (i, 0)),
        out_shape=jax.ShapeDtypeStruct((Np, 8), jnp.float32),
    )(xp, W, att_src, att_dst)


def _sc_edge_pass(srcp, dstp, dist2, ce2, T, pv, Np, rows_pad):
    STRIPE = Np // NSUB
    TOTCH = rows_pad // CH
    CBASE = TOTCH // (NCORE * NSUB)
    CEXTRA = TOTCH % (NCORE * NSUB)

    def body(src_hbm, dst_hbm, dist_hbm, ce_hbm, t_hbm, pv_hbm,
             acc_out,
             acc_sh,
             srcv0, dstv0, tsv0, adv0,
             srcv1, dstv1, tsv1, adv1,
             distv, cevv, contribv, cntv, pvv,
             gsem, ssem):
        bufs = [(srcv0, dstv0, tsv0, adv0), (srcv1, dstv1, tsv1, adv1)]
        c = lax.axis_index("c")
        s = lax.axis_index("s")
        w = c * NSUB + s
        sbase = pl.multiple_of(s * STRIPE, 128)

        pltpu.sync_copy(pv_hbm, pvv)

        iota16 = lax.iota(jnp.int32, 16)
        zf = jnp.zeros((16,), jnp.float32)

        def zbody(i, carry):
            rows = i * 16 + iota16
            for col in range(8):
                ci = jnp.full((16,), col, jnp.int32)
                plsc.store_scatter(contribv, [rows, ci], zf)
                plsc.store_scatter(cntv, [rows, ci], zf)
            return carry
        lax.fori_loop(0, K // 16, zbody, 0)

        off = 0
        rem = STRIPE
        while rem > 0:
            sz = min(rem, K)
            pltpu.sync_copy(contribv.at[pl.ds(0, sz)],
                            acc_sh.at[pl.ds(sbase + off, sz)])
            off += sz
            rem -= sz

        plsc.subcore_barrier()

        zi = jnp.zeros((16,), jnp.int32)
        oi = jnp.full((16,), 1, jnp.int32)
        v0 = plsc.load_gather(pvv, [zi])
        v1 = plsc.load_gather(pvv, [oi])

        nch = jnp.where(w < CEXTRA, CBASE + 1, CBASE)
        cstart = w * CBASE + jnp.minimum(w, CEXTRA)

        def copy_idx(t, b):
            srcv, dstv = bufs[b][0], bufs[b][1]
            row_base = (cstart + t) * CH
            pltpu.sync_copy(src_hbm.at[pl.ds(row_base, CH)], srcv)
            pltpu.sync_copy(dst_hbm.at[pl.ds(row_base, CH)], dstv)

        def copy_ea(t):
            row_base = (cstart + t) * CH
            pltpu.sync_copy(dist_hbm.at[pl.ds(row_base, CH)], distv)
            pltpu.sync_copy(ce_hbm.at[pl.ds(row_base, CH)], cevv)

        def issue_gathers(b):
            srcv, dstv, tsv, adv = bufs[b]
            for j in range(CH):
                pltpu.async_copy(
                    t_hbm.at[srcv.at[j]], tsv.at[pl.ds(j * 128, 128)], gsem)
                pltpu.async_copy(
                    t_hbm.at[dstv.at[j]], adv.at[pl.ds(j * 128, 128)], gsem)

        def wait_gathers(b):
            srcv, dstv, tsv, adv = bufs[b]
            for j in range(CH):
                pltpu.make_async_copy(
                    t_hbm.at[srcv.at[j]], tsv.at[pl.ds(j * 128, 128)],
                    gsem).wait()
                pltpu.make_async_copy(
                    t_hbm.at[dstv.at[j]], adv.at[pl.ds(j * 128, 128)],
                    gsem).wait()

        def issue_scatters(b):
            srcv, dstv, _, _ = bufs[b]
            for j in range(CH):
                pltpu.async_copy(
                    contribv.at[pl.ds(j * 128, 128)],
                    acc_sh.at[dstv.at[j]], ssem, add=True)
                pltpu.async_copy(
                    cntv.at[pl.ds(j * 128, 128)],
                    acc_sh.at[srcv.at[j]], ssem, add=True)

        def wait_scatters(b):
            srcv, dstv, _, _ = bufs[b]
            for j in range(CH):
                pltpu.make_async_copy(
                    contribv.at[pl.ds(j * 128, 128)],
                    acc_sh.at[dstv.at[j]], ssem).wait()
                pltpu.make_async_copy(
                    cntv.at[pl.ds(j * 128, 128)],
                    acc_sh.at[srcv.at[j]], ssem).wait()

        def compute(b):
            _, _, tsv, adv = bufs[b]

            def ebody(j, ecarry):
                jr = jnp.full((16,), j, jnp.int32)
                for q in range(8):
                    cq = q * 16 + iota16
                    rows = j * 128 + cq
                    asrc = plsc.load_gather(
                        tsv, [rows, jnp.full((16,), 5, jnp.int32)])
                    adst = plsc.load_gather(
                        adv, [rows, jnp.full((16,), 6, jnp.int32)])
                    dist = plsc.load_gather(distv, [jr, cq])
                    cev = plsc.load_gather(cevv, [jr, cq])
                    alpha = asrc + adst + v0 * dist + v1 * cev
                    alpha = jnp.where(alpha >= 0.0, alpha, 0.2 * alpha)
                    wgt = jnp.exp(alpha)
                    for col in range(5):
                        ci = jnp.full((16,), col, jnp.int32)
                        hc = plsc.load_gather(tsv, [rows, ci])
                        plsc.store_scatter(contribv, [rows, ci], wgt * hc)
                    plsc.store_scatter(
                        contribv, [rows, jnp.full((16,), 5, jnp.int32)], wgt)
                    below = jnp.where(dist < 0.1, 1.0, 0.0)
                    isce = jnp.where(cev == 1.0, 1.0, 0.0)
                    plsc.store_scatter(
                        cntv, [rows, jnp.full((16,), 6, jnp.int32)], below)
                    plsc.store_scatter(
                        cntv, [rows, jnp.full((16,), 7, jnp.int32)], isce)
                return ecarry
            lax.fori_loop(0, CH, ebody, 0)

        copy_idx(0, 0)
        copy_ea(0)
        issue_gathers(0)
        npairs = nch // 2
        odd = nch - npairs * 2

        def pipe_body(i, carry):
            t0 = i * 2
            t1 = t0 + 1

            @pl.when(t0 > 0)
            def _():
                wait_scatters(1)
            copy_idx(t1, 1)
            issue_gathers(1)
            wait_gathers(0)
            compute(0)
            issue_scatters(0)
            copy_ea(t1)

            wait_scatters(0)

            @pl.when(t1 + 1 < nch)
            def _():
                copy_idx(t1 + 1, 0)
                issue_gathers(0)
            wait_gathers(1)
            compute(1)
            issue_scatters(1)

            @pl.when(t1 + 1 < nch)
            def _():
                copy_ea(t1 + 1)
            return carry
        lax.fori_loop(0, npairs, pipe_body, 0)

        @pl.when(odd == 1)
        def _():
            @pl.when(npairs > 0)
            def _():
                wait_scatters(1)
            wait_gathers(0)
            compute(0)
            issue_scatters(0)
            wait_scatters(0)

        @pl.when((odd == 0) & (npairs > 0))
        def _():
            wait_scatters(1)

        plsc.subcore_barrier()

        off = 0
        rem = STRIPE
        while rem > 0:
            sz = min(rem, K)
            pltpu.sync_copy(acc_sh.at[pl.ds(sbase + off, sz)],
                            contribv.at[pl.ds(0, sz)])
            pltpu.sync_copy(contribv.at[pl.ds(0, sz)],
                            acc_out.at[c].at[pl.ds(sbase + off, sz)])
            off += sz
            rem -= sz

    mesh = plsc.VectorSubcoreMesh(
        core_axis_name="c", subcore_axis_name="s",
        num_cores=NCORE, num_subcores=NSUB)
    call = pl.kernel(
        body,
        out_type=jax.ShapeDtypeStruct((NCORE, Np, 8), jnp.float32),
        mesh=mesh,
        compiler_params=pltpu.CompilerParams(
            needs_layout_passes=False, use_tc_tiling_on_sc=False),
        scratch_types=[
            pltpu.VMEM_SHARED((Np, 8), jnp.float32),
            pltpu.VMEM((CH, 128), jnp.int32),
            pltpu.VMEM((CH, 128), jnp.int32),
            pltpu.VMEM((K, 8), jnp.float32),
            pltpu.VMEM((K, 8), jnp.float32),
            pltpu.VMEM((CH, 128), jnp.int32),
            pltpu.VMEM((CH, 128), jnp.int32),
            pltpu.VMEM((K, 8), jnp.float32),
            pltpu.VMEM((K, 8), jnp.float32),
            pltpu.VMEM((CH, 128), jnp.float32),
            pltpu.VMEM((CH, 128), jnp.float32),
            pltpu.VMEM((K, 8), jnp.float32),
            pltpu.VMEM((K, 8), jnp.float32),
            pltpu.VMEM((16,), jnp.float32),
            pltpu.SemaphoreType.DMA,
            pltpu.SemaphoreType.DMA,
        ],
    )
    return call(srcp, dstp, dist2, ce2, T, pv)


def _post_call(acc, xp, bias, W1, b1, W2, b2, Np, n):
    def body(acc_ref, x_ref, bias_ref, w1_ref, b1_ref, w2_ref,
             b2_ref, newx_ref, vel_ref, bc_ref, fr_ref, dc_ref):
        i = pl.program_id(0)
        a = acc_ref[0] + acc_ref[1]
        num = a[:, 0:5]
        den = a[:, 5:6]
        out = num / (den + 1e-16) + bias_ref[...][None, :]
        h2 = jnp.maximum(
            jnp.dot(out, w1_ref[...], preferred_element_type=jnp.float32)
            + b1_ref[...][None, :], 0.0)
        h2 = jnp.maximum(
            jnp.dot(h2, w2_ref[...], preferred_element_type=jnp.float32)
            + b2_ref[...][None, :], 0.0)
        h2 = h2 * 2.0 - 1.0
        xb = x_ref[...]
        x4 = xb[:, 4:5]
        food = jnp.where(x4 == 1.0, 1.0, 0.0)
        accv = h2 * 0.01 * food
        velo = jnp.clip(xb[:, 2:4] + accv, -0.1, 0.1)
        posn = xb[:, 0:2] + velo
        newx_ref[...] = jnp.concatenate([posn, velo, x4], axis=1)
        vel_ref[...] = velo
        rowid = lax.broadcasted_iota(jnp.int32, (BA, 1), 0) + i * BA
        valid = jnp.where(rowid < n, 1.0, 0.0)
        absx = jnp.abs(posn[:, 0:1])
        absy = jnp.abs(posn[:, 1:2])
        bx = jnp.log(absx + 1e-6) * jnp.where(absx > 1.0, 1.0, 0.0)
        by = jnp.log(absy + 1e-6) * jnp.where(absy > 1.0, 1.0, 0.0)
        bpart = jnp.sum((bx + by) * valid)
        consume = jnp.where((x4 == 0.0) & (a[:, 6:7] >= 3.0), 1.0, 0.0) * valid
        deadv = jnp.where((x4 == 1.0) & (a[:, 7:8] < 1.0), 1.0, 0.0) * valid
        fpart = jnp.sum(consume)
        dpart = jnp.sum(deadv)

        @pl.when(i == 0)
        def _():
            bc_ref[0, 0] = 0.0
            fr_ref[0, 0] = 0.0
            dc_ref[0, 0] = 0.0
        bc_ref[0, 0] += bpart
        fr_ref[0, 0] += fpart
        dc_ref[0, 0] += dpart

    return pl.pallas_call(
        body,
        grid=(Np // BA,),
        in_specs=[
            pl.BlockSpec((2, BA, 8), lambda i: (0, i, 0)),
            pl.BlockSpec((BA, 5), lambda i: (i, 0)),
            pl.BlockSpec((5,), lambda i: (0,)),
            pl.BlockSpec((5, 5), lambda i: (0, 0)),
            pl.BlockSpec((5,), lambda i: (0,)),
            pl.BlockSpec((5, 2), lambda i: (0, 0)),
            pl.BlockSpec((2,), lambda i: (0,)),
        ],
        out_specs=[
            pl.BlockSpec((BA, 5), lambda i: (i, 0)),
            pl.BlockSpec((BA, 2), lambda i: (i, 0)),
            pl.BlockSpec((1, 1), lambda i: (0, 0), memory_space=pltpu.SMEM),
            pl.BlockSpec((1, 1), lambda i: (0, 0), memory_space=pltpu.SMEM),
            pl.BlockSpec((1, 1), lambda i: (0, 0), memory_space=pltpu.SMEM),
        ],
        out_shape=[
            jax.ShapeDtypeStruct((Np, 5), jnp.float32),
            jax.ShapeDtypeStruct((Np, 2), jnp.float32),
            jax.ShapeDtypeStruct((1, 1), jnp.float32),
            jax.ShapeDtypeStruct((1, 1), jnp.float32),
            jax.ShapeDtypeStruct((1, 1), jnp.float32),
        ],
    )(acc, xp, bias, W1, b1, W2, b2)


def kernel(x, edge_index, edge_attr, W, att_src, att_dst, W_edge, att_edge,
           bias, W1, b1, W2, b2):
    n = x.shape[0]
    e = edge_index.shape[1]
    Np = ((n + 1 + BA - 1) // BA) * BA
    rows = (e + 127) // 128
    rows_pad = ((rows + CH - 1) // CH) * CH
    Ep = rows_pad * 128
    pad_e = Ep - e

    xp = jnp.zeros((Np, 5), jnp.float32).at[:n, :].set(x)
    src = edge_index[0]
    dst = edge_index[1]
    dummy = jnp.full((pad_e,), n, jnp.int32)
    srcp = jnp.concatenate([src, dummy]).reshape(rows_pad, 128)
    dstp = jnp.concatenate([dst, dummy]).reshape(rows_pad, 128)
    zpad = jnp.zeros((pad_e,), jnp.float32)
    dist2 = jnp.concatenate(
        [edge_attr[:, 0].astype(jnp.float32), zpad]).reshape(rows_pad, 128)
    ce2 = jnp.concatenate(
        [edge_attr[:, 1].astype(jnp.float32), zpad]).reshape(rows_pad, 128)
    v = W_edge @ att_edge
    pv = jnp.concatenate([v, jnp.zeros((14,), jnp.float32)]).astype(jnp.float32)

    T = _prep_call(xp, W, att_src, att_dst, Np)
    acc = _sc_edge_pass(srcp, dstp, dist2, ce2, T, pv, Np, rows_pad)
    newx, vel, bc, fr, dc = _post_call(acc, xp, bias, W1, b1, W2, b2, Np, n)
    return (newx[:n], vel[:n], bc.reshape(()), fr.reshape(()),
            dc.reshape(()))


# speedup vs baseline: 186.2884x; 1.0488x over previous
import functools

import jax
import jax.numpy as jnp
from jax import lax
from jax.experimental import pallas as pl
from jax.experimental.pallas import tpu as pltpu
from jax.experimental.pallas import tpu_sc as plsc

BA = 2048
CH = 8
K = CH * 128
NSUB = 16
NCORE = 2


def _prep_call(xp, W, att_src, att_dst, Np):
    def body(x_ref, w_ref, asr_ref, adr_ref, t_ref):
        xb = x_ref[...]
        h = jnp.dot(xb, w_ref[...], preferred_element_type=jnp.float32)
        asrc = jnp.sum(h * asr_ref[...][None, :], axis=1, keepdims=True)
        adst = jnp.sum(h * adr_ref[...][None, :], axis=1, keepdims=True)
        t_ref[...] = jnp.concatenate(
            [h, asrc, adst, jnp.zeros((BA, 1), jnp.float32)], axis=1)

    return pl.pallas_call(
        body,
        grid=(Np // BA,),
        in_specs=[
            pl.BlockSpec((BA, 5), lambda i: (i, 0)),
            pl.BlockSpec((5, 5), lambda i: (0, 0)),
            pl.BlockSpec((5,), lambda i: (0,)),
            pl.BlockSpec((5,), lambda i: (0,)),
        ],
        out_specs=pl.BlockSpec((BA, 8), lambda i: (i, 0)),
        out_shape=jax.ShapeDtypeStruct((Np, 8), jnp.float32),
    )(xp, W, att_src, att_dst)


def _sc_edge_pass(eidx2, ea3, T, pv, Np, rows_pad):
    STRIPE = Np // NSUB
    TOTCH = rows_pad // CH
    CBASE = TOTCH // (NCORE * NSUB)
    CEXTRA = TOTCH % (NCORE * NSUB)

    DOFF = rows_pad

    def body(ei_hbm, ea_hbm, t_hbm, pv_hbm,
             acc_out,
             acc_sh,
             srcv0, dstv0, tsv0, adv0,
             srcv1, dstv1, tsv1, adv1,
             distv, cevv, contribv, cntv, pvv,
             gsem, ssem):
        bufs = [(srcv0, dstv0, tsv0, adv0), (srcv1, dstv1, tsv1, adv1)]
        c = lax.axis_index("c")
        s = lax.axis_index("s")
        w = c * NSUB + s
        sbase = pl.multiple_of(s * STRIPE, 128)

        pltpu.sync_copy(pv_hbm, pvv)

        iota16 = lax.iota(jnp.int32, 16)
        zf = jnp.zeros((16,), jnp.float32)

        def zbody(i, carry):
            rows = i * 16 + iota16
            for col in range(8):
                ci = jnp.full((16,), col, jnp.int32)
                plsc.store_scatter(contribv, [rows, ci], zf)
                plsc.store_scatter(cntv, [rows, ci], zf)
            return carry
        lax.fori_loop(0, K // 16, zbody, 0)

        off = 0
        rem = STRIPE
        while rem > 0:
            sz = min(rem, K)
            pltpu.sync_copy(contribv.at[pl.ds(0, sz)],
                            acc_sh.at[pl.ds(sbase + off, sz)])
            off += sz
            rem -= sz

        plsc.subcore_barrier()

        zi = jnp.zeros((16,), jnp.int32)
        oi = jnp.full((16,), 1, jnp.int32)
        v0 = plsc.load_gather(pvv, [zi])
        v1 = plsc.load_gather(pvv, [oi])

        nch = jnp.where(w < CEXTRA, CBASE + 1, CBASE)
        cstart = w * CBASE + jnp.minimum(w, CEXTRA)

        def copy_idx(t, b):
            srcv, dstv = bufs[b][0], bufs[b][1]
            row_base = (cstart + t) * CH
            pltpu.sync_copy(ei_hbm.at[pl.ds(row_base, CH)], srcv)
            pltpu.sync_copy(ei_hbm.at[pl.ds(DOFF + row_base, CH)], dstv)

        def copy_ea(t):
            row_base = (cstart + t) * CH
            pltpu.sync_copy(ea_hbm.at[pl.ds(row_base, CH), 0], distv)
            pltpu.sync_copy(ea_hbm.at[pl.ds(row_base, CH), 1], cevv)

        def issue_gathers(b):
            srcv, dstv, tsv, adv = bufs[b]
            for j in range(CH):
                pltpu.async_copy(
                    t_hbm.at[srcv.at[j]], tsv.at[pl.ds(j * 128, 128)], gsem)
                pltpu.async_copy(
                    t_hbm.at[dstv.at[j]], adv.at[pl.ds(j * 128, 128)], gsem)

        def wait_gathers(b):
            srcv, dstv, tsv, adv = bufs[b]
            for j in range(CH):
                pltpu.make_async_copy(
                    t_hbm.at[srcv.at[j]], tsv.at[pl.ds(j * 128, 128)],
                    gsem).wait()
                pltpu.make_async_copy(
                    t_hbm.at[dstv.at[j]], adv.at[pl.ds(j * 128, 128)],
                    gsem).wait()

        def issue_scatters(b):
            srcv, dstv, _, _ = bufs[b]
            for j in range(CH):
                pltpu.async_copy(
                    contribv.at[pl.ds(j * 128, 128)],
                    acc_sh.at[dstv.at[j]], ssem, add=True)
                pltpu.async_copy(
                    cntv.at[pl.ds(j * 128, 128)],
                    acc_sh.at[srcv.at[j]], ssem, add=True)

        def wait_scatters(b):
            srcv, dstv, _, _ = bufs[b]
            for j in range(CH):
                pltpu.make_async_copy(
                    contribv.at[pl.ds(j * 128, 128)],
                    acc_sh.at[dstv.at[j]], ssem).wait()
                pltpu.make_async_copy(
                    cntv.at[pl.ds(j * 128, 128)],
                    acc_sh.at[srcv.at[j]], ssem).wait()

        def compute(b):
            _, _, tsv, adv = bufs[b]

            def ebody(j, ecarry):
                jr = jnp.full((16,), j, jnp.int32)
                for q in range(8):
                    cq = q * 16 + iota16
                    rows = j * 128 + cq
                    asrc = plsc.load_gather(
                        tsv, [rows, jnp.full((16,), 5, jnp.int32)])
                    adst = plsc.load_gather(
                        adv, [rows, jnp.full((16,), 6, jnp.int32)])
                    dist = plsc.load_gather(distv, [jr, cq])
                    cev = plsc.load_gather(cevv, [jr, cq])
                    alpha = asrc + adst + v0 * dist + v1 * cev
                    alpha = jnp.where(alpha >= 0.0, alpha, 0.2 * alpha)
                    wgt = jnp.exp(alpha)
                    for col in range(5):
                        ci = jnp.full((16,), col, jnp.int32)
                        hc = plsc.load_gather(tsv, [rows, ci])
                        plsc.store_scatter(contribv, [rows, ci], wgt * hc)
                    plsc.store_scatter(
                        contribv, [rows, jnp.full((16,), 5, jnp.int32)], wgt)
                    below = jnp.where(dist < 0.1, 1.0, 0.0)
                    isce = jnp.where(cev == 1.0, 1.0, 0.0)
                    plsc.store_scatter(
                        cntv, [rows, jnp.full((16,), 6, jnp.int32)], below)
                    plsc.store_scatter(
                        cntv, [rows, jnp.full((16,), 7, jnp.int32)], isce)
                return ecarry
            lax.fori_loop(0, CH, ebody, 0)

        copy_idx(0, 0)
        copy_ea(0)
        issue_gathers(0)
        npairs = nch // 2
        odd = nch - npairs * 2

        def pipe_body(i, carry):
            t0 = i * 2
            t1 = t0 + 1

            @pl.when(t0 > 0)
            def _():
                wait_scatters(1)
            copy_idx(t1, 1)
            issue_gathers(1)
            wait_gathers(0)
            compute(0)
            issue_scatters(0)
            copy_ea(t1)

            wait_scatters(0)

            @pl.when(t1 + 1 < nch)
            def _():
                copy_idx(t1 + 1, 0)
                issue_gathers(0)
            wait_gathers(1)
            compute(1)
            issue_scatters(1)

            @pl.when(t1 + 1 < nch)
            def _():
                copy_ea(t1 + 1)
            return carry
        lax.fori_loop(0, npairs, pipe_body, 0)

        @pl.when(odd == 1)
        def _():
            @pl.when(npairs > 0)
            def _():
                wait_scatters(1)
            wait_gathers(0)
            compute(0)
            issue_scatters(0)
            wait_scatters(0)

        @pl.when((odd == 0) & (npairs > 0))
        def _():
            wait_scatters(1)

        plsc.subcore_barrier()

        off = 0
        rem = STRIPE
        while rem > 0:
            sz = min(rem, K)
            pltpu.sync_copy(acc_sh.at[pl.ds(sbase + off, sz)],
                            contribv.at[pl.ds(0, sz)])
            pltpu.sync_copy(contribv.at[pl.ds(0, sz)],
                            acc_out.at[c].at[pl.ds(sbase + off, sz)])
            off += sz
            rem -= sz

    mesh = plsc.VectorSubcoreMesh(
        core_axis_name="c", subcore_axis_name="s",
        num_cores=NCORE, num_subcores=NSUB)
    call = pl.kernel(
        body,
        out_type=jax.ShapeDtypeStruct((NCORE, Np, 8), jnp.float32),
        mesh=mesh,
        compiler_params=pltpu.CompilerParams(
            needs_layout_passes=False, use_tc_tiling_on_sc=False),
        scratch_types=[
            pltpu.VMEM_SHARED((Np, 8), jnp.float32),
            pltpu.VMEM((CH, 128), jnp.int32),
            pltpu.VMEM((CH, 128), jnp.int32),
            pltpu.VMEM((K, 8), jnp.float32),
            pltpu.VMEM((K, 8), jnp.float32),
            pltpu.VMEM((CH, 128), jnp.int32),
            pltpu.VMEM((CH, 128), jnp.int32),
            pltpu.VMEM((K, 8), jnp.float32),
            pltpu.VMEM((K, 8), jnp.float32),
            pltpu.VMEM((CH, 128), jnp.float32),
            pltpu.VMEM((CH, 128), jnp.float32),
            pltpu.VMEM((K, 8), jnp.float32),
            pltpu.VMEM((K, 8), jnp.float32),
            pltpu.VMEM((16,), jnp.float32),
            pltpu.SemaphoreType.DMA,
            pltpu.SemaphoreType.DMA,
        ],
    )
    return call(eidx2, ea3, T, pv)


def _post_call(acc, xp, bias, W1, b1, W2, b2, Np, n):
    def body(acc_ref, x_ref, bias_ref, w1_ref, b1_ref, w2_ref,
             b2_ref, newx_ref, vel_ref, bc_ref, fr_ref, dc_ref):
        i = pl.program_id(0)
        a = acc_ref[0] + acc_ref[1]
        num = a[:, 0:5]
        den = a[:, 5:6]
        out = num / (den + 1e-16) + bias_ref[...][None, :]
        h2 = jnp.maximum(
            jnp.dot(out, w1_ref[...], preferred_element_type=jnp.float32)
            + b1_ref[...][None, :], 0.0)
        h2 = jnp.maximum(
            jnp.dot(h2, w2_ref[...], preferred_element_type=jnp.float32)
            + b2_ref[...][None, :], 0.0)
        h2 = h2 * 2.0 - 1.0
        xb = x_ref[...]
        x4 = xb[:, 4:5]
        food = jnp.where(x4 == 1.0, 1.0, 0.0)
        accv = h2 * 0.01 * food
        velo = jnp.clip(xb[:, 2:4] + accv, -0.1, 0.1)
        posn = xb[:, 0:2] + velo
        newx_ref[...] = jnp.concatenate([posn, velo, x4], axis=1)
        vel_ref[...] = velo
        rowid = lax.broadcasted_iota(jnp.int32, (BA, 1), 0) + i * BA
        valid = jnp.where(rowid < n, 1.0, 0.0)
        absx = jnp.abs(posn[:, 0:1])
        absy = jnp.abs(posn[:, 1:2])
        bx = jnp.log(absx + 1e-6) * jnp.where(absx > 1.0, 1.0, 0.0)
        by = jnp.log(absy + 1e-6) * jnp.where(absy > 1.0, 1.0, 0.0)
        bpart = jnp.sum((bx + by) * valid)
        consume = jnp.where((x4 == 0.0) & (a[:, 6:7] >= 3.0), 1.0, 0.0) * valid
        deadv = jnp.where((x4 == 1.0) & (a[:, 7:8] < 1.0), 1.0, 0.0) * valid
        fpart = jnp.sum(consume)
        dpart = jnp.sum(deadv)

        @pl.when(i == 0)
        def _():
            bc_ref[0, 0] = 0.0
            fr_ref[0, 0] = 0.0
            dc_ref[0, 0] = 0.0
        bc_ref[0, 0] += bpart
        fr_ref[0, 0] += fpart
        dc_ref[0, 0] += dpart

    return pl.pallas_call(
        body,
        grid=(Np // BA,),
        in_specs=[
            pl.BlockSpec((2, BA, 8), lambda i: (0, i, 0)),
            pl.BlockSpec((BA, 5), lambda i: (i, 0)),
            pl.BlockSpec((5,), lambda i: (0,)),
            pl.BlockSpec((5, 5), lambda i: (0, 0)),
            pl.BlockSpec((5,), lambda i: (0,)),
            pl.BlockSpec((5, 2), lambda i: (0, 0)),
            pl.BlockSpec((2,), lambda i: (0,)),
        ],
        out_specs=[
            pl.BlockSpec((BA, 5), lambda i: (i, 0)),
            pl.BlockSpec((BA, 2), lambda i: (i, 0)),
            pl.BlockSpec((1, 1), lambda i: (0, 0), memory_space=pltpu.SMEM),
            pl.BlockSpec((1, 1), lambda i: (0, 0), memory_space=pltpu.SMEM),
            pl.BlockSpec((1, 1), lambda i: (0, 0), memory_space=pltpu.SMEM),
        ],
        out_shape=[
            jax.ShapeDtypeStruct((Np, 5), jnp.float32),
            jax.ShapeDtypeStruct((Np, 2), jnp.float32),
            jax.ShapeDtypeStruct((1, 1), jnp.float32),
            jax.ShapeDtypeStruct((1, 1), jnp.float32),
            jax.ShapeDtypeStruct((1, 1), jnp.float32),
        ],
    )(acc, xp, bias, W1, b1, W2, b2)


def kernel(x, edge_index, edge_attr, W, att_src, att_dst, W_edge, att_edge,
           bias, W1, b1, W2, b2):
    n = x.shape[0]
    e = edge_index.shape[1]
    Np = ((n + 1 + BA - 1) // BA) * BA
    rows = (e + 127) // 128
    rows_pad = ((rows + CH - 1) // CH) * CH
    Ep = rows_pad * 128
    pad_e = Ep - e

    xp = jnp.zeros((Np, 5), jnp.float32).at[:n, :].set(x)
    if pad_e == 0:
        eidx2 = edge_index.reshape(2 * rows_pad, 128)
        ea3 = edge_attr.astype(jnp.float32).reshape(
            rows_pad, 128, 2).transpose(0, 2, 1)
    else:
        dummy = jnp.full((pad_e,), n, jnp.int32)
        srcp = jnp.concatenate([edge_index[0], dummy])
        dstp = jnp.concatenate([edge_index[1], dummy])
        eidx2 = jnp.concatenate([srcp, dstp]).reshape(2 * rows_pad, 128)
        eap = jnp.concatenate(
            [edge_attr.astype(jnp.float32),
             jnp.zeros((pad_e, 2), jnp.float32)], axis=0)
        ea3 = eap.reshape(rows_pad, 128, 2).transpose(0, 2, 1)
    v = W_edge @ att_edge
    pv = jnp.concatenate([v, jnp.zeros((14,), jnp.float32)]).astype(jnp.float32)

    T = _prep_call(xp, W, att_src, att_dst, Np)
    acc = _sc_edge_pass(eidx2, ea3, T, pv, Np, rows_pad)
    newx, vel, bc, fr, dc = _post_call(acc, xp, bias, W1, b1, W2, b2, Np, n)
    return (newx[:n], vel[:n], bc.reshape(()), fr.reshape(()),
            dc.reshape(()))
